# baseline (speedup 1.0000x reference)
import functools

import jax
import jax.numpy as jnp
from jax import lax
from jax.experimental import pallas as pl
from jax.experimental.pallas import tpu as pltpu
from jax.experimental.pallas import tpu_sc as plsc

B = 16384
D = 32
V = 1000000
NC = 2
NS = 16
NW = NC * NS
B_PER_W = B // NW

RBLK = 2048


def _repack_body(*refs):
    ins, outs = refs[:4], refs[4:]
    r = lax.broadcasted_iota(jnp.int32, (D, D), 0)
    c = lax.broadcasted_iota(jnp.int32, (D, D), 1)
    eye = jnp.where(r == c, 1.0, 0.0).astype(jnp.float32)
    for x_ref, y_ref in zip(ins, outs):
        y_ref[...] = lax.dot_general(x_ref[...], eye, (((0,), (0,)), ((), ())),
                                     preferred_element_type=jnp.float32)


def _repack(tables_t):
    return pl.pallas_call(
        _repack_body,
        grid=((V + RBLK - 1) // RBLK,),
        in_specs=[pl.BlockSpec((D, RBLK), lambda j: (0, j))] * 4,
        out_specs=[pl.BlockSpec((RBLK, D), lambda j: (j, 0))] * 4,
        out_shape=[jax.ShapeDtypeStruct((V, D), jnp.float32)] * 4,
    )(*tables_t)


def _sc_gather_body(uidx_hbm, iidx_hbm, ug_hbm, ig_hbm, um_hbm, im_hbm,
                    out_ug, out_ig, out_um, out_im,
                    uidx_v, iidx_v, r_ug, r_ig, r_um, r_im, sem):
    wid = lax.axis_index("s") * NC + lax.axis_index("c")
    base = wid * B_PER_W
    pltpu.sync_copy(uidx_hbm.at[pl.ds(base, B_PER_W)], uidx_v)
    pltpu.sync_copy(iidx_hbm.at[pl.ds(base, B_PER_W)], iidx_v)
    c1 = pltpu.async_copy(ug_hbm.at[uidx_v], r_ug, sem)
    c2 = pltpu.async_copy(ig_hbm.at[iidx_v], r_ig, sem)
    c3 = pltpu.async_copy(um_hbm.at[uidx_v], r_um, sem)
    c4 = pltpu.async_copy(im_hbm.at[iidx_v], r_im, sem)
    c1.wait(); c2.wait(); c3.wait(); c4.wait()
    pltpu.sync_copy(r_ug, out_ug.at[pl.ds(base, B_PER_W)])
    pltpu.sync_copy(r_ig, out_ig.at[pl.ds(base, B_PER_W)])
    pltpu.sync_copy(r_um, out_um.at[pl.ds(base, B_PER_W)])
    pltpu.sync_copy(r_im, out_im.at[pl.ds(base, B_PER_W)])


_sc_gather = functools.partial(
    pl.kernel,
    out_type=[jax.ShapeDtypeStruct((B, D), jnp.float32)] * 4,
    mesh=plsc.VectorSubcoreMesh(core_axis_name="c", subcore_axis_name="s"),
    compiler_params=pltpu.CompilerParams(use_tc_tiling_on_sc=False),
    scratch_types=[
        pltpu.VMEM((B_PER_W,), jnp.int32),
        pltpu.VMEM((B_PER_W,), jnp.int32),
        pltpu.VMEM((B_PER_W, D), jnp.float32),
        pltpu.VMEM((B_PER_W, D), jnp.float32),
        pltpu.VMEM((B_PER_W, D), jnp.float32),
        pltpu.VMEM((B_PER_W, D), jnp.float32),
        pltpu.SemaphoreType.DMA,
    ],
)(_sc_gather_body)


def _dot_t(x, w):
    return lax.dot_general(x, w, (((1,), (1,)), ((), ())),
                           preferred_element_type=jnp.float32)


def _tc_dense_body(ug_ref, ig_ref, um_ref, im_ref,
                   w1a_ref, w1b_ref, b1_ref, w2_ref, b2_ref, w3_ref, b3_ref,
                   wpa_ref, wpb_ref, bp_ref, out_ref):
    mf = ug_ref[...] * ig_ref[...]
    h = _dot_t(um_ref[...], w1a_ref[...]) + _dot_t(im_ref[...], w1b_ref[...])
    h = jnp.maximum(h + b1_ref[...], 0.0)
    h = jnp.maximum(_dot_t(h, w2_ref[...]) + b2_ref[...], 0.0)
    h = jnp.maximum(_dot_t(h, w3_ref[...]) + b3_ref[...], 0.0)
    out_ref[...] = _dot_t(mf, wpa_ref[...]) + _dot_t(h, wpb_ref[...]) + bp_ref[...]


def kernel(user_indices, item_indices, U_gmf, I_gmf, U_mlp, I_mlp,
           W1, b1, W2, b2, W3, b3, Wp, bp):
    tables = _repack([t.T for t in (U_gmf, I_gmf, U_mlp, I_mlp)])
    ug, ig, um, im = _sc_gather(user_indices, item_indices, *tables)
    w1a, w1b = W1[:, :D], W1[:, D:]
    wpa, wpb = Wp[:, :D], Wp[:, D:]
    pred = pl.pallas_call(
        _tc_dense_body,
        out_shape=jax.ShapeDtypeStruct((B, 1), jnp.float32),
    )(ug, ig, um, im,
      w1a, w1b, b1.reshape(1, -1), W2, b2.reshape(1, -1),
      W3, b3.reshape(1, -1), wpa, wpb, bp.reshape(1, 1))
    return pred.reshape(-1)


# speedup vs baseline: 4.5763x; 4.5763x over previous
import functools

import jax
import jax.numpy as jnp
from jax import lax
from jax.experimental import pallas as pl
from jax.experimental.pallas import tpu as pltpu
from jax.experimental.pallas import tpu_sc as plsc

B = 16384
D = 32
NC = 2
NS = 16
NW = NC * NS
B_PER_W = B // NW
K = 4
HALF = B_PER_W // 2


def _extract(sb, g, lane, col):
    rows = lax.iota(jnp.int32, 16)
    lanev = jnp.broadcast_to(lane, (16,))
    colv = jnp.broadcast_to(col, (16,))
    lo = plsc.load_gather(sb, [rows, lanev])
    hi = plsc.load_gather(sb, [rows + 16, lanev])
    plsc.store_scatter(g, [rows, colv], lo)
    plsc.store_scatter(g, [rows + 16, colv], hi)


def _sc_gather_body(uidx_hbm, iidx_hbm, ugT, igT, umT, imT,
                    out_ug, out_ig, out_um, out_im, *scratch):
    u_s, i_s = scratch[0], scratch[1]
    slots = scratch[2:2 + 4 * K]
    g_ug, g_ig, g_um, g_im = scratch[2 + 4 * K:6 + 4 * K]
    sem = scratch[6 + 4 * K]
    wid = lax.axis_index("s") * NC + lax.axis_index("c")
    base = wid * B_PER_W
    pltpu.sync_copy(uidx_hbm.at[pl.ds(base, B_PER_W)], u_s.at[pl.ds(0, B_PER_W)])
    pltpu.sync_copy(iidx_hbm.at[pl.ds(base, B_PER_W)], i_s.at[pl.ds(0, B_PER_W)])

    for h in range(2):
        def chunk(c):
            start = h * HALF + c * K
            uvec = u_s[pl.ds(start, 16)]
            ivec = i_s[pl.ds(start, 16)]
            copies = []
            for j in range(K):
                u = uvec[j]
                v = ivec[j]
                u128 = pl.multiple_of((u >> 7) << 7, 128)
                v128 = pl.multiple_of((v >> 7) << 7, 128)
                copies.append(pltpu.async_copy(
                    ugT.at[:, pl.ds(u128, 128)], slots[4 * j + 0], sem))
                copies.append(pltpu.async_copy(
                    umT.at[:, pl.ds(u128, 128)], slots[4 * j + 1], sem))
                copies.append(pltpu.async_copy(
                    igT.at[:, pl.ds(v128, 128)], slots[4 * j + 2], sem))
                copies.append(pltpu.async_copy(
                    imT.at[:, pl.ds(v128, 128)], slots[4 * j + 3], sem))
            for cp in copies:
                cp.wait()
            for j in range(K):
                col = c * K + j
                ul = uvec[j] & 127
                vl = ivec[j] & 127
                _extract(slots[4 * j + 0], g_ug, ul, col)
                _extract(slots[4 * j + 1], g_um, ul, col)
                _extract(slots[4 * j + 2], g_ig, vl, col)
                _extract(slots[4 * j + 3], g_im, vl, col)

        pl.loop(0, HALF // K)(chunk)
        off = base + h * HALF
        pltpu.sync_copy(g_ug, out_ug.at[:, pl.ds(off, HALF)])
        pltpu.sync_copy(g_ig, out_ig.at[:, pl.ds(off, HALF)])
        pltpu.sync_copy(g_um, out_um.at[:, pl.ds(off, HALF)])
        pltpu.sync_copy(g_im, out_im.at[:, pl.ds(off, HALF)])


_sc_gather = functools.partial(
    pl.kernel,
    out_type=[jax.ShapeDtypeStruct((D, B), jnp.float32)] * 4,
    mesh=plsc.VectorSubcoreMesh(core_axis_name="c", subcore_axis_name="s"),
    compiler_params=pltpu.CompilerParams(needs_layout_passes=False),
    scratch_types=(
        [pltpu.VMEM((B_PER_W + 16,), jnp.int32)] * 2
        + [pltpu.VMEM((D, 128), jnp.float32)] * (4 * K)
        + [pltpu.VMEM((D, HALF), jnp.float32)] * 4
        + [pltpu.SemaphoreType.DMA]
    ),
)(_sc_gather_body)


def _mm(w, x):
    return lax.dot_general(w, x, (((1,), (0,)), ((), ())),
                           preferred_element_type=jnp.float32)


def _tc_dense_body(ug_ref, ig_ref, um_ref, im_ref,
                   w1a_ref, w1b_ref, b1_ref, w2_ref, b2_ref, w3_ref, b3_ref,
                   wpa_ref, wpb_ref, bp_ref, out_ref):
    mf = ug_ref[...] * ig_ref[...]
    h = _mm(w1a_ref[...], um_ref[...]) + _mm(w1b_ref[...], im_ref[...])
    h = jnp.maximum(h + b1_ref[...], 0.0)
    h = jnp.maximum(_mm(w2_ref[...], h) + b2_ref[...], 0.0)
    h = jnp.maximum(_mm(w3_ref[...], h) + b3_ref[...], 0.0)
    out_ref[...] = _mm(wpa_ref[...], mf) + _mm(wpb_ref[...], h) + bp_ref[...]


def kernel(user_indices, item_indices, U_gmf, I_gmf, U_mlp, I_mlp,
           W1, b1, W2, b2, W3, b3, Wp, bp):
    ug, ig, um, im = _sc_gather(user_indices, item_indices,
                                U_gmf.T, I_gmf.T, U_mlp.T, I_mlp.T)
    w1a, w1b = W1[:, :D], W1[:, D:]
    wpa, wpb = Wp[:, :D], Wp[:, D:]
    pred = pl.pallas_call(
        _tc_dense_body,
        out_shape=jax.ShapeDtypeStruct((1, B), jnp.float32),
    )(ug, ig, um, im,
      w1a, w1b, b1.reshape(-1, 1), W2, b2.reshape(-1, 1),
      W3, b3.reshape(-1, 1), wpa, wpb, bp.reshape(1, 1))
    return pred.reshape(-1)


# speedup vs baseline: 4.8231x; 1.0539x over previous
import functools

import jax
import jax.numpy as jnp
from jax import lax
from jax.experimental import pallas as pl
from jax.experimental.pallas import tpu as pltpu
from jax.experimental.pallas import tpu_sc as plsc

B = 16384
D = 32
NC = 2
NS = 16
NW = NC * NS
B_PER_W = B // NW
K = 4
HALF = B_PER_W // 2


def _extract(sb, g, lane, col):
    rows = lax.iota(jnp.int32, 16)
    lanev = jnp.broadcast_to(lane, (16,))
    colv = jnp.broadcast_to(col, (16,))
    lo = plsc.load_gather(sb, [rows, lanev])
    hi = plsc.load_gather(sb, [rows + 16, lanev])
    plsc.store_scatter(g, [rows, colv], lo)
    plsc.store_scatter(g, [rows + 16, colv], hi)


KP = 2


def _sc_gather_body(uidx_hbm, iidx_hbm, ugT, igT, umT, imT,
                    out_ug, out_ig, out_um, out_im, *scratch):
    u_s, i_s = scratch[0], scratch[1]
    slots = scratch[2:2 + 4 * K]
    g_ug, g_ig, g_um, g_im = scratch[2 + 4 * K:6 + 4 * K]
    sem_a, sem_b = scratch[6 + 4 * K], scratch[7 + 4 * K]
    grp = (slots[:4 * KP], slots[4 * KP:])
    sems = (sem_a, sem_b)
    wid = lax.axis_index("s") * NC + lax.axis_index("c")
    base = wid * B_PER_W
    pltpu.sync_copy(uidx_hbm.at[pl.ds(base, B_PER_W)], u_s.at[pl.ds(0, B_PER_W)])
    pltpu.sync_copy(iidx_hbm.at[pl.ds(base, B_PER_W)], i_s.at[pl.ds(0, B_PER_W)])

    def fire(g, start):
        uvec = u_s[pl.ds(start, 16)]
        ivec = i_s[pl.ds(start, 16)]
        for j in range(KP):
            u = uvec[j]
            v = ivec[j]
            u128 = pl.multiple_of((u >> 7) << 7, 128)
            v128 = pl.multiple_of((v >> 7) << 7, 128)
            pltpu.async_copy(ugT.at[:, pl.ds(u128, 128)], grp[g][4 * j + 0], sems[g])
            pltpu.async_copy(umT.at[:, pl.ds(u128, 128)], grp[g][4 * j + 1], sems[g])
            pltpu.async_copy(igT.at[:, pl.ds(v128, 128)], grp[g][4 * j + 2], sems[g])
            pltpu.async_copy(imT.at[:, pl.ds(v128, 128)], grp[g][4 * j + 3], sems[g])

    def drain_extract(g, start, h):
        for s in grp[g]:
            pltpu.make_async_copy(ugT.at[:, pl.ds(0, 128)], s, sems[g]).wait()
        uvec = u_s[pl.ds(start, 16)]
        ivec = i_s[pl.ds(start, 16)]
        for j in range(KP):
            col = start - h * HALF + j
            ul = uvec[j] & 127
            vl = ivec[j] & 127
            _extract(grp[g][4 * j + 0], g_ug, ul, col)
            _extract(grp[g][4 * j + 1], g_um, ul, col)
            _extract(grp[g][4 * j + 2], g_ig, vl, col)
            _extract(grp[g][4 * j + 3], g_im, vl, col)

    for h in range(2):
        def body(c):
            i0 = h * HALF + c * 2 * KP
            fire(0, i0)

            @pl.when(c > 0)
            def _():
                drain_extract(1, i0 - KP, h)

            fire(1, i0 + KP)
            drain_extract(0, i0, h)

        pl.loop(0, HALF // (2 * KP))(body)
        drain_extract(1, h * HALF + HALF - KP, h)
        off = base + h * HALF
        pltpu.sync_copy(g_ug, out_ug.at[:, pl.ds(off, HALF)])
        pltpu.sync_copy(g_ig, out_ig.at[:, pl.ds(off, HALF)])
        pltpu.sync_copy(g_um, out_um.at[:, pl.ds(off, HALF)])
        pltpu.sync_copy(g_im, out_im.at[:, pl.ds(off, HALF)])


_sc_gather = functools.partial(
    pl.kernel,
    out_type=[jax.ShapeDtypeStruct((D, B), jnp.float32)] * 4,
    mesh=plsc.VectorSubcoreMesh(core_axis_name="c", subcore_axis_name="s"),
    compiler_params=pltpu.CompilerParams(needs_layout_passes=False),
    scratch_types=(
        [pltpu.VMEM((B_PER_W + 16,), jnp.int32)] * 2
        + [pltpu.VMEM((D, 128), jnp.float32)] * (4 * K)
        + [pltpu.VMEM((D, HALF), jnp.float32)] * 4
        + [pltpu.SemaphoreType.DMA] * 2
    ),
)(_sc_gather_body)


def _mm(w, x):
    return lax.dot_general(w, x, (((1,), (0,)), ((), ())),
                           preferred_element_type=jnp.float32)


def _tc_dense_body(ug_ref, ig_ref, um_ref, im_ref,
                   w1a_ref, w1b_ref, b1_ref, w2_ref, b2_ref, w3_ref, b3_ref,
                   wpa_ref, wpb_ref, bp_ref, out_ref):
    mf = ug_ref[...] * ig_ref[...]
    h = _mm(w1a_ref[...], um_ref[...]) + _mm(w1b_ref[...], im_ref[...])
    h = jnp.maximum(h + b1_ref[...], 0.0)
    h = jnp.maximum(_mm(w2_ref[...], h) + b2_ref[...], 0.0)
    h = jnp.maximum(_mm(w3_ref[...], h) + b3_ref[...], 0.0)
    out_ref[...] = _mm(wpa_ref[...], mf) + _mm(wpb_ref[...], h) + bp_ref[...]


def kernel(user_indices, item_indices, U_gmf, I_gmf, U_mlp, I_mlp,
           W1, b1, W2, b2, W3, b3, Wp, bp):
    ug, ig, um, im = _sc_gather(user_indices, item_indices,
                                U_gmf.T, I_gmf.T, U_mlp.T, I_mlp.T)
    w1a, w1b = W1[:, :D], W1[:, D:]
    wpa, wpb = Wp[:, :D], Wp[:, D:]
    pred = pl.pallas_call(
        _tc_dense_body,
        out_shape=jax.ShapeDtypeStruct((1, B), jnp.float32),
    )(ug, ig, um, im,
      w1a, w1b, b1.reshape(-1, 1), W2, b2.reshape(-1, 1),
      W3, b3.reshape(-1, 1), wpa, wpb, bp.reshape(1, 1))
    return pred.reshape(-1)


# speedup vs baseline: 4.9853x; 1.0336x over previous
import functools

import jax
import jax.numpy as jnp
from jax import lax
from jax.experimental import pallas as pl
from jax.experimental.pallas import tpu as pltpu
from jax.experimental.pallas import tpu_sc as plsc

B = 16384
D = 32
NC = 2
NS = 16
NW = NC * NS
B_PER_W = B // NW
K = 4
HALF = B_PER_W // 2


def _extract(sb, g, lane, col):
    rows = lax.iota(jnp.int32, 16)
    lanev = jnp.broadcast_to(lane, (16,))
    colv = jnp.broadcast_to(col, (16,))
    lo = plsc.load_gather(sb, [rows, lanev])
    hi = plsc.load_gather(sb, [rows + 16, lanev])
    plsc.store_scatter(g, [rows, colv], lo)
    plsc.store_scatter(g, [rows + 16, colv], hi)


NG = 4


def _sc_gather_body(uidx_hbm, iidx_hbm, ugT, igT, umT, imT,
                    out_ug, out_ig, out_um, out_im, *scratch):
    u_s, i_s = scratch[0], scratch[1]
    slots = scratch[2:2 + 4 * NG]
    g_ug, g_ig, g_um, g_im = scratch[2 + 4 * NG:6 + 4 * NG]
    sems = scratch[6 + 4 * NG:6 + 5 * NG]
    grp = tuple(slots[4 * g:4 * g + 4] for g in range(NG))
    wid = lax.axis_index("s") * NC + lax.axis_index("c")
    base = wid * B_PER_W
    pltpu.sync_copy(uidx_hbm.at[pl.ds(base, B_PER_W)], u_s.at[pl.ds(0, B_PER_W)])
    pltpu.sync_copy(iidx_hbm.at[pl.ds(base, B_PER_W)], i_s.at[pl.ds(0, B_PER_W)])

    def fire(g, i):
        u = u_s[pl.ds(i, 16)][0]
        v = i_s[pl.ds(i, 16)][0]
        u128 = pl.multiple_of((u >> 7) << 7, 128)
        v128 = pl.multiple_of((v >> 7) << 7, 128)
        pltpu.async_copy(ugT.at[:, pl.ds(u128, 128)], grp[g][0], sems[g])
        pltpu.async_copy(umT.at[:, pl.ds(u128, 128)], grp[g][1], sems[g])
        pltpu.async_copy(igT.at[:, pl.ds(v128, 128)], grp[g][2], sems[g])
        pltpu.async_copy(imT.at[:, pl.ds(v128, 128)], grp[g][3], sems[g])

    def drain_extract(g, i, h):
        for s in grp[g]:
            pltpu.make_async_copy(ugT.at[:, pl.ds(0, 128)], s, sems[g]).wait()
        ul = u_s[pl.ds(i, 16)][0] & 127
        vl = i_s[pl.ds(i, 16)][0] & 127
        col = i - h * HALF
        _extract(grp[g][0], g_ug, ul, col)
        _extract(grp[g][1], g_um, ul, col)
        _extract(grp[g][2], g_ig, vl, col)
        _extract(grp[g][3], g_im, vl, col)

    for h in range(2):
        def body(c):
            i0 = h * HALF + c * NG
            fire(0, i0)

            @pl.when(c > 0)
            def _():
                drain_extract(2, i0 - 2, h)

            fire(1, i0 + 1)

            @pl.when(c > 0)
            def _():
                drain_extract(3, i0 - 1, h)

            fire(2, i0 + 2)
            drain_extract(0, i0, h)
            fire(3, i0 + 3)
            drain_extract(1, i0 + 1, h)

        pl.loop(0, HALF // NG)(body)
        drain_extract(2, h * HALF + HALF - 2, h)
        drain_extract(3, h * HALF + HALF - 1, h)
        off = base + h * HALF
        pltpu.sync_copy(g_ug, out_ug.at[:, pl.ds(off, HALF)])
        pltpu.sync_copy(g_ig, out_ig.at[:, pl.ds(off, HALF)])
        pltpu.sync_copy(g_um, out_um.at[:, pl.ds(off, HALF)])
        pltpu.sync_copy(g_im, out_im.at[:, pl.ds(off, HALF)])


_sc_gather = functools.partial(
    pl.kernel,
    out_type=[jax.ShapeDtypeStruct((D, B), jnp.float32)] * 4,
    mesh=plsc.VectorSubcoreMesh(core_axis_name="c", subcore_axis_name="s"),
    compiler_params=pltpu.CompilerParams(needs_layout_passes=False),
    scratch_types=(
        [pltpu.VMEM((B_PER_W + 16,), jnp.int32)] * 2
        + [pltpu.VMEM((D, 128), jnp.float32)] * (4 * NG)
        + [pltpu.VMEM((D, HALF), jnp.float32)] * 4
        + [pltpu.SemaphoreType.DMA] * NG
    ),
)(_sc_gather_body)


def _mm(w, x):
    return lax.dot_general(w, x, (((1,), (0,)), ((), ())),
                           preferred_element_type=jnp.float32)


def _tc_dense_body(ug_ref, ig_ref, um_ref, im_ref,
                   w1a_ref, w1b_ref, b1_ref, w2_ref, b2_ref, w3_ref, b3_ref,
                   wpa_ref, wpb_ref, bp_ref, out_ref):
    mf = ug_ref[...] * ig_ref[...]
    h = _mm(w1a_ref[...], um_ref[...]) + _mm(w1b_ref[...], im_ref[...])
    h = jnp.maximum(h + b1_ref[...], 0.0)
    h = jnp.maximum(_mm(w2_ref[...], h) + b2_ref[...], 0.0)
    h = jnp.maximum(_mm(w3_ref[...], h) + b3_ref[...], 0.0)
    out_ref[...] = _mm(wpa_ref[...], mf) + _mm(wpb_ref[...], h) + bp_ref[...]


def kernel(user_indices, item_indices, U_gmf, I_gmf, U_mlp, I_mlp,
           W1, b1, W2, b2, W3, b3, Wp, bp):
    ug, ig, um, im = _sc_gather(user_indices, item_indices,
                                U_gmf.T, I_gmf.T, U_mlp.T, I_mlp.T)
    w1a, w1b = W1[:, :D], W1[:, D:]
    wpa, wpb = Wp[:, :D], Wp[:, D:]
    pred = pl.pallas_call(
        _tc_dense_body,
        out_shape=jax.ShapeDtypeStruct((1, B), jnp.float32),
    )(ug, ig, um, im,
      w1a, w1b, b1.reshape(-1, 1), W2, b2.reshape(-1, 1),
      W3, b3.reshape(-1, 1), wpa, wpb, bp.reshape(1, 1))
    return pred.reshape(-1)
